# baseline (speedup 1.0000x reference)
import functools

import jax
import jax.numpy as jnp
from jax import lax
from jax.experimental import pallas as pl
from jax.experimental.pallas import tpu as pltpu
from jax.experimental.pallas import tpu_sc as plsc

B = 32
N = 4096
P = 256
L = 16
RADIX_BITS = 11
NBINS = 1 << RADIX_BITS
NVREG = N // L
CHUNK = 64
NCHUNKS = N // CHUNK
NBUF = 6
CSHIFT = CHUNK.bit_length() - 1


def _sortkey(bits):
    m = bits >> 31
    u = bits ^ (m | jnp.int32(-2147483648))
    return ~u


def _digit(sk, shift):
    return (sk >> shift) & jnp.int32(NBINS - 1)


def _make_kernel():
    mesh = plsc.VectorSubcoreMesh(core_axis_name="c", subcore_axis_name="s")
    info = plsc.get_sparse_core_info()
    nc = info.num_cores

    @functools.partial(
        pl.kernel,
        out_type=jax.ShapeDtypeStruct((B, N, P), jnp.float32),
        mesh=mesh,
        compiler_params=pltpu.CompilerParams(needs_layout_passes=False),
        scratch_types=[
            pltpu.VMEM((N,), jnp.int32),
            pltpu.VMEM((N,), jnp.int32),
            pltpu.VMEM((N,), jnp.int32),
            pltpu.VMEM((N,), jnp.int32),
            pltpu.VMEM((N,), jnp.int32),
            pltpu.VMEM((NBINS,), jnp.int32),
            pltpu.VMEM((NCHUNKS, CHUNK), jnp.int32),
            pltpu.VMEM((NBUF, CHUNK, P), jnp.float32),
            pltpu.SemaphoreType.DMA((NBUF,)),
            pltpu.SemaphoreType.DMA((NBUF,)),
        ],
    )
    def body(x_hbm, out_hbm,
             kf, ka, kb, va, vb, hist, idx2, bufs, gsems, ssems):
        wid = lax.axis_index("s") * nc + lax.axis_index("c")
        xb = x_hbm.at[0, wid]
        ob = out_hbm.at[wid]

        col127 = jnp.full((L,), P - 1 - 128, jnp.int32)
        zero = jnp.zeros((L,), jnp.int32)

        def fire_key_dma(c):
            return pltpu.async_copy(
                xb.at[pl.ds(c * CHUNK, CHUNK), 128:256],
                bufs.at[c % NBUF].at[:, 0:128], gsems.at[c % NBUF])

        kd = [None] * NCHUNKS
        for c in range(NBUF):
            kd[c] = fire_key_dma(c)

        def zero_body0(i, carry):
            hist[pl.ds(i * L, L)] = zero
            return carry
        lax.fori_loop(0, NBINS // L, zero_body0, 0, unroll=8)

        for c in range(NCHUNKS):
            kd[c].wait()
            bi = c % NBUF
            for j in range(CHUNK // L):
                rows = lax.iota(jnp.int32, L) + (j * L)
                v = plsc.load_gather(bufs.at[bi], [rows, col127])
                sk = _sortkey(plsc.bitcast(v, jnp.int32))
                kf[pl.ds(c * CHUNK + j * L, L)] = sk
                d0 = sk & jnp.int32(NBINS - 1)
                cnt, is_last = plsc.scan_count(d0)
                plsc.addupdate_scatter(hist, [d0], cnt, mask=is_last)
            if c + NBUF < NCHUNKS:
                kd[c + NBUF] = fire_key_dma(c + NBUF)
        kfc = kf

        def radix_pass(shift, src_k, src_v, dst_k, dst_v, first, last_pass):
            if not first:
                def zero_body(i, carry):
                    hist[pl.ds(i * L, L)] = zero
                    return carry
                lax.fori_loop(0, NBINS // L, zero_body, 0, unroll=8)

                @plsc.parallel_loop(0, NVREG, 1, unroll=16)
                def _hist_body(i):
                    sk = src_k[pl.ds(i * L, L)]
                    d = _digit(sk, shift)
                    cnt, is_last = plsc.scan_count(d)
                    plsc.addupdate_scatter(hist, [d], cnt, mask=is_last)

            def scan_body(i, carry):
                h = hist[pl.ds(i * L, L)]
                c = plsc.cumsum(h)
                hist[pl.ds(i * L, L)] = c - h + carry
                return carry + jnp.sum(h)
            lax.fori_loop(0, NBINS // L, scan_body, jnp.int32(0), unroll=2)

            def perm_body(i, carry):
                if first:
                    sk = kfc[pl.ds(i * L, L)]
                    v = lax.iota(jnp.int32, L) + (i * L)
                else:
                    sk = src_k[pl.ds(i * L, L)]
                    v = src_v[pl.ds(i * L, L)]
                d = _digit(sk, shift)
                cnt, is_last = plsc.scan_count(d)
                base = plsc.load_gather(hist, [d])
                pos = base + cnt - 1
                if last_pass:
                    plsc.store_scatter(
                        dst_v, [pos >> CSHIFT, pos & (CHUNK - 1)], v)
                else:
                    plsc.store_scatter(dst_k, [pos], sk)
                    plsc.store_scatter(dst_v, [pos], v)
                plsc.addupdate_scatter(hist, [d], cnt, mask=is_last)
                return carry
            lax.fori_loop(0, NVREG, perm_body, 0, unroll=4)

        radix_pass(0, None, None, kb, vb, True, False)
        radix_pass(RADIX_BITS, kb, vb, ka, va, False, False)
        radix_pass(2 * RADIX_BITS, ka, va, None, idx2, False, True)

        gd = [None] * NCHUNKS
        sd = [None] * NCHUNKS
        for c in range(NCHUNKS + 1):
            if c < NCHUNKS:
                bi = c % NBUF
                if c >= NBUF:
                    sd[c - NBUF].wait()
                gd[c] = pltpu.async_copy(
                    xb.at[idx2.at[c]], bufs.at[bi], gsems.at[bi])
            d = c - 1
            if d >= 0:
                gd[d].wait()
                sd[d] = pltpu.async_copy(
                    bufs.at[d % NBUF],
                    ob.at[pl.ds(d * CHUNK, CHUNK)],
                    ssems.at[d % NBUF])
        for d in range(NCHUNKS - NBUF, NCHUNKS):
            sd[d].wait()

    return body


_sorted_gather = _make_kernel()


@jax.jit
def kernel(x):
    return _sorted_gather(x)
